# baseline (speedup 1.0000x reference)
import jax
import jax.numpy as jnp
from jax.experimental import pallas as pl
from jax.experimental.pallas import tpu as pltpu

_CI = 7
_CO = 7


def kernel(r_idx, kv):
    b, p2, w2, c_kv = kv.shape
    topk = r_idx.shape[2]
    qk = p2 * topk
    sub = (w2 * c_kv) // 128

    rc = p2 // _CI
    oc = qk // _CO

    kv_r = kv.reshape(b, p2, sub, 128)
    flat_idx = r_idx.reshape(b, qk).astype(jnp.int32)

    def body(idx_ref, kv_hbm, out_hbm, in_buf, out_buf, in_sems, out_sems):
        def start_in(bi):
            slot = bi % 2
            for c in range(_CI):
                pltpu.make_async_copy(
                    kv_hbm.at[bi, pl.ds(c * rc, rc)],
                    in_buf.at[slot, pl.ds(c * rc, rc)],
                    in_sems.at[slot, c],
                ).start()

        def wait_in(bi):
            slot = bi % 2
            for c in range(_CI):
                pltpu.make_async_copy(
                    kv_hbm.at[bi, pl.ds(c * rc, rc)],
                    in_buf.at[slot, pl.ds(c * rc, rc)],
                    in_sems.at[slot, c],
                ).wait()

        def start_out(bi):
            slot = bi % 2
            for c in range(_CO):
                pltpu.make_async_copy(
                    out_buf.at[slot, pl.ds(c * oc, oc)],
                    out_hbm.at[bi, pl.ds(c * oc, oc)],
                    out_sems.at[slot, c],
                ).start()

        def wait_out(bi):
            slot = bi % 2
            for c in range(_CO):
                pltpu.make_async_copy(
                    out_buf.at[slot, pl.ds(c * oc, oc)],
                    out_hbm.at[bi, pl.ds(c * oc, oc)],
                    out_sems.at[slot, c],
                ).wait()

        start_in(0)

        def step(bi, carry):
            slot = bi % 2

            @pl.when(bi + 1 < b)
            def _():
                start_in(bi + 1)

            wait_in(bi)

            @pl.when(bi >= 2)
            def _():
                wait_out(bi - 2)

            for j in range(qk):
                out_buf[slot, j] = in_buf[slot, idx_ref[bi, j]]

            start_out(bi)
            return carry

        jax.lax.fori_loop(0, b, step, 0)
        wait_out(b - 2)
        wait_out(b - 1)

    grid_spec = pltpu.PrefetchScalarGridSpec(
        num_scalar_prefetch=1,
        grid=(1,),
        in_specs=[pl.BlockSpec(memory_space=pl.ANY)],
        out_specs=pl.BlockSpec(memory_space=pl.ANY),
        scratch_shapes=[
            pltpu.VMEM((2, p2, sub, 128), jnp.float32),
            pltpu.VMEM((2, qk, sub, 128), jnp.float32),
            pltpu.SemaphoreType.DMA((2, _CI)),
            pltpu.SemaphoreType.DMA((2, _CO)),
        ],
    )

    out = pl.pallas_call(
        body,
        grid_spec=grid_spec,
        out_shape=jax.ShapeDtypeStruct((b, qk, sub, 128), kv.dtype),
        compiler_params=pltpu.CompilerParams(
            vmem_limit_bytes=100 * 1024 * 1024,
        ),
    )(flat_idx, kv_r)

    return out.reshape(b, p2, topk, w2, c_kv)


# speedup vs baseline: 1.0022x; 1.0022x over previous
import jax
import jax.numpy as jnp
from jax.experimental import pallas as pl
from jax.experimental.pallas import tpu as pltpu

_CI = 7
_CO = 7


def kernel(r_idx, kv):
    b, p2, w2, c_kv = kv.shape
    topk = r_idx.shape[2]
    qk = p2 * topk
    sub = (w2 * c_kv) // 128

    rc = p2 // _CI
    oc = qk // _CO

    kv_r = kv.reshape(b, p2, sub, 128)
    flat_idx = r_idx.reshape(b, qk).astype(jnp.int32)

    def body(idx_ref, kv_hbm, out_hbm,
             in0, in1, ob0, ob1, is0, is1, os0, os1):

        def start_in(bi, buf, sems):
            for c in range(_CI):
                pltpu.make_async_copy(
                    kv_hbm.at[bi, pl.ds(c * rc, rc)],
                    buf.at[pl.ds(c * rc, rc)],
                    sems.at[c],
                ).start()

        def wait_in(bi, buf, sems):
            for c in range(_CI):
                pltpu.make_async_copy(
                    kv_hbm.at[bi, pl.ds(c * rc, rc)],
                    buf.at[pl.ds(c * rc, rc)],
                    sems.at[c],
                ).wait()

        def start_out(bi, buf, sems):
            for c in range(_CO):
                pltpu.make_async_copy(
                    buf.at[pl.ds(c * oc, oc)],
                    out_hbm.at[bi, pl.ds(c * oc, oc)],
                    sems.at[c],
                ).start()

        def wait_out(bi, buf, sems):
            for c in range(_CO):
                pltpu.make_async_copy(
                    buf.at[pl.ds(c * oc, oc)],
                    out_hbm.at[bi, pl.ds(c * oc, oc)],
                    sems.at[c],
                ).wait()

        def gather(bi, src, dst):
            for j in range(qk):
                dst[j] = src[idx_ref[bi, j]]

        start_in(0, in0, is0)

        def step(it, carry):
            b0 = 2 * it
            b1 = b0 + 1

            start_in(b1, in1, is1)
            wait_in(b0, in0, is0)

            @pl.when(it >= 1)
            def _():
                wait_out(b0 - 2, ob0, os0)

            gather(b0, in0, ob0)
            start_out(b0, ob0, os0)

            @pl.when(b1 + 1 < b)
            def _():
                start_in(b1 + 1, in0, is0)

            wait_in(b1, in1, is1)

            @pl.when(it >= 1)
            def _():
                wait_out(b1 - 2, ob1, os1)

            gather(b1, in1, ob1)
            start_out(b1, ob1, os1)
            return carry

        jax.lax.fori_loop(0, b // 2, step, 0)
        wait_out(b - 2, ob0, os0)
        wait_out(b - 1, ob1, os1)

    grid_spec = pltpu.PrefetchScalarGridSpec(
        num_scalar_prefetch=1,
        grid=(1,),
        in_specs=[pl.BlockSpec(memory_space=pl.ANY)],
        out_specs=pl.BlockSpec(memory_space=pl.ANY),
        scratch_shapes=[
            pltpu.VMEM((p2, sub, 128), jnp.float32),
            pltpu.VMEM((p2, sub, 128), jnp.float32),
            pltpu.VMEM((qk, sub, 128), jnp.float32),
            pltpu.VMEM((qk, sub, 128), jnp.float32),
            pltpu.SemaphoreType.DMA((_CI,)),
            pltpu.SemaphoreType.DMA((_CI,)),
            pltpu.SemaphoreType.DMA((_CO,)),
            pltpu.SemaphoreType.DMA((_CO,)),
        ],
    )

    out = pl.pallas_call(
        body,
        grid_spec=grid_spec,
        out_shape=jax.ShapeDtypeStruct((b, qk, sub, 128), kv.dtype),
        compiler_params=pltpu.CompilerParams(
            vmem_limit_bytes=100 * 1024 * 1024,
        ),
    )(flat_idx, kv_r)

    return out.reshape(b, p2, topk, w2, c_kv)
